# baseline (speedup 1.0000x reference)
import functools

import jax
import jax.numpy as jnp
from jax import lax
from jax.experimental import pallas as pl
from jax.experimental.pallas import tpu as pltpu
from jax.experimental.pallas import tpu_sc as plsc

_N = 10000
_E = 320000
_D = 128
_DH = _D // 2

_NC = 2
_NS = 16
_EC = _E // _NS
_B = 128
_NB = 160
_ECP = _NB * _B
_NP = 10240
_RPT = _NP // _NS
_DD = 16
_BR = 512


def _mesh():
    return plsc.VectorSubcoreMesh(core_axis_name="c", subcore_axis_name="s")


def _deg_kernel(interpret=False):
    return pl.kernel(
        _deg_body,
        out_type=jax.ShapeDtypeStruct((_NC, _NP, _DD), jnp.float32),
        mesh=_mesh(),
        compiler_params=pltpu.CompilerParams(use_tc_tiling_on_sc=False),
        scratch_types=[
            pltpu.VMEM((_NB, _B), jnp.int32),
            pltpu.VMEM((_B, _DD), jnp.float32),
            pltpu.VMEM((_B, _DD), jnp.float32),
            pltpu.VMEM_SHARED((_NP, _DD), jnp.float32),
        ],
        interpret=interpret,
    )


def _deg_body(dst_hbm, out_hbm, dst_v, ones_v, zero_v, acc):
    c = lax.axis_index("c")
    s = lax.axis_index("s")
    pltpu.sync_copy(dst_hbm.at[s], dst_v)

    ov = jnp.ones((16,), jnp.float32)
    zv = jnp.zeros((16,), jnp.float32)

    def _fill(i, carry):
        ones_v[i, pl.ds(0, _DD)] = ov
        zero_v[i, pl.ds(0, _DD)] = zv
        return carry

    lax.fori_loop(0, _B, _fill, 0)
    rbase = s * _RPT
    for r in range(_RPT // _B):
        pltpu.sync_copy(zero_v, acc.at[pl.ds(rbase + r * _B, _B)])
    plsc.subcore_barrier()

    half = _NB // 2

    def _body(b, carry):
        pltpu.sync_copy(ones_v, acc.at[dst_v.at[c * half + b]], add=True)
        return carry

    lax.fori_loop(0, half, _body, 0)
    plsc.subcore_barrier()
    pltpu.sync_copy(acc.at[pl.ds(rbase, _RPT)], out_hbm.at[c, pl.ds(rbase, _RPT)])


def _scatter_kernel(interpret=False):
    return pl.kernel(
        _scatter_body,
        out_type=jax.ShapeDtypeStruct((_NC, _NP, _DH), jnp.float32),
        mesh=_mesh(),
        compiler_params=pltpu.CompilerParams(use_tc_tiling_on_sc=False),
        scratch_types=[
            pltpu.VMEM((_ECP,), jnp.int32),
            pltpu.VMEM((_NB, _B), jnp.int32),
            pltpu.VMEM((2, 2 * _B, _DH), jnp.float32),
            pltpu.VMEM_SHARED((_NP, _DH), jnp.float32),
            [pltpu.SemaphoreType.DMA] * 2,
            [pltpu.SemaphoreType.DMA] * 2,
        ],
        interpret=interpret,
    )


def _scatter_body(y_hbm, src_hbm, dst_hbm, out_hbm, src_v, dst_v, stage, acc, gsem, ssem):
    c = lax.axis_index("c")
    s = lax.axis_index("s")
    pltpu.sync_copy(src_hbm.at[s], src_v)
    pltpu.sync_copy(dst_hbm.at[s], dst_v)

    def _xform(i, carry):
        v = src_v[pl.ds(i * 16, 16)]
        src_v[pl.ds(i * 16, 16)] = v * 2 + c
        return carry

    lax.fori_loop(0, _ECP // 16, _xform, 0)

    zv = jnp.zeros((16,), jnp.float32)

    def _zrow(i, carry):
        for k in range(2):
            for j in range(_DH // 16):
                stage[k, i, pl.ds(j * 16, 16)] = zv
        return carry

    lax.fori_loop(0, 2 * _B, _zrow, 0)
    rbase = s * _RPT
    for r in range(_RPT // (2 * _B)):
        pltpu.sync_copy(stage.at[0], acc.at[pl.ds(rbase + r * 2 * _B, 2 * _B)])
    rem = _RPT % (2 * _B)
    if rem:
        pltpu.sync_copy(
            stage.at[0, pl.ds(0, rem)], acc.at[pl.ds(rbase + _RPT - rem, rem)]
        )
    plsc.subcore_barrier()

    def _g_start(sb, k):
        bb = sb * 2
        pltpu.make_async_copy(
            y_hbm.at[src_v.at[pl.ds(bb * _B, _B)]],
            stage.at[k, pl.ds(0, _B)], gsem[k],
        ).start()
        pltpu.make_async_copy(
            y_hbm.at[src_v.at[pl.ds((bb + 1) * _B, _B)]],
            stage.at[k, pl.ds(_B, _B)], gsem[k],
        ).start()

    def _g_wait(k):
        pltpu.make_async_copy(
            y_hbm.at[pl.ds(0, 2 * _B)], stage.at[k], gsem[k]
        ).wait()

    def _s_start(sb, k):
        bb = sb * 2
        pltpu.async_copy(
            stage.at[k, pl.ds(0, _B)], acc.at[dst_v.at[bb]], ssem[k], add=True)
        pltpu.async_copy(
            stage.at[k, pl.ds(_B, _B)], acc.at[dst_v.at[bb + 1]], ssem[k],
            add=True)

    def _s_wait(k):
        pltpu.make_async_copy(
            y_hbm.at[pl.ds(0, 2 * _B)], stage.at[k], ssem[k]
        ).wait()

    _g_start(0, 0)
    _g_wait(0); _s_start(0, 0); _g_start(1, 1)

    def _body(g, carry):
        sb0 = 2 * g + 1
        _g_wait(1); _s_start(sb0, 1); _s_wait(0); _g_start(sb0 + 1, 0)
        _g_wait(0); _s_start(sb0 + 1, 0); _s_wait(1); _g_start(sb0 + 2, 1)
        return carry

    lax.fori_loop(0, (_NB // 2) // 2 - 1, _body, 0)
    _g_wait(1); _s_start(_NB // 2 - 1, 1); _s_wait(0); _s_wait(1)

    plsc.subcore_barrier()
    pltpu.sync_copy(acc.at[pl.ds(rbase, _RPT)], out_hbm.at[c, pl.ds(rbase, _RPT)])


def _dis_from(pd):
    deg = pd[0, :, 0:1] + pd[1, :, 0:1] + 1.0
    return lax.rsqrt(deg)


def _halves(p):
    return jnp.concatenate((p[0], p[1]), axis=-1)


def _tc1_body(pd_ref, x_ref, w_ref, y_ref):
    dis = _dis_from(pd_ref[...])
    y_ref[...] = jnp.dot(
        x_ref[...], w_ref[...], preferred_element_type=jnp.float32
    ) * dis


def _tc2_body(pd_ref, p_ref, y1_ref, b1_ref, w_ref, y2_ref):
    dis = _dis_from(pd_ref[...])
    pre = (_halves(p_ref[...]) + y1_ref[...]) * dis + b1_ref[...]
    h = jnp.maximum(pre, 0.0)
    y2_ref[...] = jnp.dot(
        h, w_ref[...], preferred_element_type=jnp.float32
    ) * dis


def _tc3_body(pd_ref, p_ref, y2_ref, b2_ref, o_ref):
    dis = _dis_from(pd_ref[...])
    o_ref[...] = (_halves(p_ref[...]) + y2_ref[...]) * dis + b2_ref[...]


_pd_spec = pl.BlockSpec((2, _BR, _DD), lambda i: (0, i, 0))
_row_spec = pl.BlockSpec((_BR, _D), lambda i: (i, 0))
_p_spec = pl.BlockSpec((2, _BR, _DH), lambda i: (0, i, 0))
_w_spec = pl.BlockSpec((_D, _D), lambda i: (0, 0))
_b_spec = pl.BlockSpec((1, _D), lambda i: (0, 0))
_rows_out = jax.ShapeDtypeStruct((_NP, _D), jnp.float32)
_grid = (_NP // _BR,)


def _tc1(pdeg, xp, w1):
    return pl.pallas_call(
        _tc1_body,
        grid=_grid,
        in_specs=[_pd_spec, _row_spec, _w_spec],
        out_specs=_row_spec,
        out_shape=_rows_out,
    )(pdeg, xp, w1)


def _tc2(pdeg, p1, y1, b1, w2):
    return pl.pallas_call(
        _tc2_body,
        grid=_grid,
        in_specs=[_pd_spec, _p_spec, _row_spec, _b_spec, _w_spec],
        out_specs=_row_spec,
        out_shape=_rows_out,
    )(pdeg, p1, y1, b1, w2)


def _tc3(pdeg, p2, y2, b2):
    return pl.pallas_call(
        _tc3_body,
        grid=_grid,
        in_specs=[_pd_spec, _p_spec, _row_spec, _b_spec],
        out_specs=_row_spec,
        out_shape=_rows_out,
    )(pdeg, p2, y2, b2)


def kernel(x, edge_index, W1, b1, W2, b2):
    src = edge_index[0].astype(jnp.int32).reshape(_NS, _EC)
    dst = edge_index[1].astype(jnp.int32).reshape(_NS, _EC)
    pad = _ECP - _EC
    srcp = jnp.pad(src, ((0, 0), (0, pad)), constant_values=0)
    dstp = jnp.pad(dst, ((0, 0), (0, pad)), constant_values=_N)
    dstp = dstp.reshape(_NS, _NB, _B)
    xp = jnp.pad(x, ((0, _NP - _N), (0, 0)))

    pdeg = _deg_kernel()(dstp)
    y1 = _tc1(pdeg, xp, W1)
    p1 = _scatter_kernel()(y1.reshape(_NP * 2, _DH), srcp, dstp)
    y2 = _tc2(pdeg, p1, y1, b1.reshape(1, _D), W2)
    p2 = _scatter_kernel()(y2.reshape(_NP * 2, _DH), srcp, dstp)
    out = _tc3(pdeg, p2, y2, b2.reshape(1, _D))
    return out[:_N]


# speedup vs baseline: 1.0003x; 1.0003x over previous
import functools

import jax
import jax.numpy as jnp
from jax import lax
from jax.experimental import pallas as pl
from jax.experimental.pallas import tpu as pltpu
from jax.experimental.pallas import tpu_sc as plsc

_N = 10000
_E = 320000
_D = 128
_DH = _D // 2

_NC = 2
_NS = 16
_EC = _E // _NS
_B = 128
_NB = 160
_ECP = _NB * _B
_NP = 10240
_RPT = _NP // _NS
_DD = 16
_BR = 512


def _mesh():
    return plsc.VectorSubcoreMesh(core_axis_name="c", subcore_axis_name="s")


def _deg_kernel(interpret=False):
    return pl.kernel(
        _deg_body,
        out_type=jax.ShapeDtypeStruct((_NC, _NP, _DD), jnp.float32),
        mesh=_mesh(),
        compiler_params=pltpu.CompilerParams(use_tc_tiling_on_sc=False),
        scratch_types=[
            pltpu.VMEM((_NB, _B), jnp.int32),
            pltpu.VMEM((_B, _DD), jnp.float32),
            pltpu.VMEM((_B, _DD), jnp.float32),
            pltpu.VMEM_SHARED((_NP, _DD), jnp.float32),
        ],
        interpret=interpret,
    )


def _deg_body(dst_hbm, out_hbm, dst_v, ones_v, zero_v, acc):
    c = lax.axis_index("c")
    s = lax.axis_index("s")
    pltpu.sync_copy(dst_hbm.at[s], dst_v)

    ov = jnp.ones((16,), jnp.float32)
    zv = jnp.zeros((16,), jnp.float32)

    def _fill(i, carry):
        ones_v[i, pl.ds(0, _DD)] = ov
        zero_v[i, pl.ds(0, _DD)] = zv
        return carry

    lax.fori_loop(0, _B, _fill, 0)
    rbase = s * _RPT
    for r in range(_RPT // _B):
        pltpu.sync_copy(zero_v, acc.at[pl.ds(rbase + r * _B, _B)])
    plsc.subcore_barrier()

    half = _NB // 2

    def _body(b, carry):
        pltpu.sync_copy(ones_v, acc.at[dst_v.at[c * half + b]], add=True)
        return carry

    lax.fori_loop(0, half, _body, 0)
    plsc.subcore_barrier()
    pltpu.sync_copy(acc.at[pl.ds(rbase, _RPT)], out_hbm.at[c, pl.ds(rbase, _RPT)])


def _scatter_kernel(interpret=False):
    return pl.kernel(
        _scatter_body,
        out_type=jax.ShapeDtypeStruct((_NC, _NP, _DH), jnp.float32),
        mesh=_mesh(),
        compiler_params=pltpu.CompilerParams(use_tc_tiling_on_sc=False),
        scratch_types=[
            pltpu.VMEM((_ECP,), jnp.int32),
            pltpu.VMEM((_NB, _B), jnp.int32),
            pltpu.VMEM((2, 2 * _B, _DH), jnp.float32),
            pltpu.VMEM_SHARED((_NP, _DH), jnp.float32),
            [pltpu.SemaphoreType.DMA] * 2,
            [pltpu.SemaphoreType.DMA] * 2,
        ],
        interpret=interpret,
    )


def _scatter_body(y_hbm, src_hbm, dst_hbm, out_hbm, src_v, dst_v, stage, acc, gsem, ssem):
    c = lax.axis_index("c")
    s = lax.axis_index("s")
    pltpu.sync_copy(src_hbm.at[s], src_v)
    pltpu.sync_copy(dst_hbm.at[s], dst_v)

    def _xform(i, carry):
        v = src_v[pl.ds(i * 16, 16)]
        src_v[pl.ds(i * 16, 16)] = v * 2 + c
        return carry

    lax.fori_loop(0, _ECP // 16, _xform, 0)

    zv = jnp.zeros((16,), jnp.float32)

    def _zrow(i, carry):
        for k in range(2):
            for j in range(_DH // 16):
                stage[k, i, pl.ds(j * 16, 16)] = zv
        return carry

    lax.fori_loop(0, 2 * _B, _zrow, 0)
    rbase = s * _RPT
    for r in range(_RPT // (2 * _B)):
        pltpu.sync_copy(stage.at[0], acc.at[pl.ds(rbase + r * 2 * _B, 2 * _B)])
    rem = _RPT % (2 * _B)
    if rem:
        pltpu.sync_copy(
            stage.at[0, pl.ds(0, rem)], acc.at[pl.ds(rbase + _RPT - rem, rem)]
        )
    plsc.subcore_barrier()

    def _g_start(sb, k):
        bb = sb * 2
        pltpu.make_async_copy(
            y_hbm.at[src_v.at[pl.ds(bb * _B, _B)]],
            stage.at[k, pl.ds(0, _B)], gsem[k],
        ).start()
        pltpu.make_async_copy(
            y_hbm.at[src_v.at[pl.ds((bb + 1) * _B, _B)]],
            stage.at[k, pl.ds(_B, _B)], gsem[k],
        ).start()

    def _g_wait(k):
        pltpu.make_async_copy(
            y_hbm.at[pl.ds(0, 2 * _B)], stage.at[k], gsem[k]
        ).wait()

    def _s_start(sb, k):
        bb = sb * 2
        pltpu.async_copy(
            stage.at[k, pl.ds(0, _B)], acc.at[dst_v.at[bb]], ssem[k],
            priority=1, add=True)
        pltpu.async_copy(
            stage.at[k, pl.ds(_B, _B)], acc.at[dst_v.at[bb + 1]], ssem[k],
            priority=1, add=True)

    def _s_wait(k):
        pltpu.make_async_copy(
            y_hbm.at[pl.ds(0, 2 * _B)], stage.at[k], ssem[k]
        ).wait()

    _g_start(0, 0)
    _g_wait(0); _s_start(0, 0); _g_start(1, 1)

    def _body(g, carry):
        sb0 = 2 * g + 1
        _g_wait(1); _s_start(sb0, 1); _s_wait(0); _g_start(sb0 + 1, 0)
        _g_wait(0); _s_start(sb0 + 1, 0); _s_wait(1); _g_start(sb0 + 2, 1)
        return carry

    lax.fori_loop(0, (_NB // 2) // 2 - 1, _body, 0)
    _g_wait(1); _s_start(_NB // 2 - 1, 1); _s_wait(0); _s_wait(1)

    plsc.subcore_barrier()
    pltpu.sync_copy(acc.at[pl.ds(rbase, _RPT)], out_hbm.at[c, pl.ds(rbase, _RPT)])


def _dis_from(pd):
    deg = pd[0, :, 0:1] + pd[1, :, 0:1] + 1.0
    return lax.rsqrt(deg)


def _halves(p):
    return jnp.concatenate((p[0], p[1]), axis=-1)


def _tc1_body(pd_ref, x_ref, w_ref, y_ref):
    dis = _dis_from(pd_ref[...])
    y_ref[...] = jnp.dot(
        x_ref[...], w_ref[...], preferred_element_type=jnp.float32
    ) * dis


def _tc2_body(pd_ref, p_ref, y1_ref, b1_ref, w_ref, y2_ref):
    dis = _dis_from(pd_ref[...])
    pre = (_halves(p_ref[...]) + y1_ref[...]) * dis + b1_ref[...]
    h = jnp.maximum(pre, 0.0)
    y2_ref[...] = jnp.dot(
        h, w_ref[...], preferred_element_type=jnp.float32
    ) * dis


def _tc3_body(pd_ref, p_ref, y2_ref, b2_ref, o_ref):
    dis = _dis_from(pd_ref[...])
    o_ref[...] = (_halves(p_ref[...]) + y2_ref[...]) * dis + b2_ref[...]


_pd_spec = pl.BlockSpec((2, _BR, _DD), lambda i: (0, i, 0))
_row_spec = pl.BlockSpec((_BR, _D), lambda i: (i, 0))
_p_spec = pl.BlockSpec((2, _BR, _DH), lambda i: (0, i, 0))
_w_spec = pl.BlockSpec((_D, _D), lambda i: (0, 0))
_b_spec = pl.BlockSpec((1, _D), lambda i: (0, 0))
_rows_out = jax.ShapeDtypeStruct((_NP, _D), jnp.float32)
_grid = (_NP // _BR,)


def _tc1(pdeg, xp, w1):
    return pl.pallas_call(
        _tc1_body,
        grid=_grid,
        in_specs=[_pd_spec, _row_spec, _w_spec],
        out_specs=_row_spec,
        out_shape=_rows_out,
    )(pdeg, xp, w1)


def _tc2(pdeg, p1, y1, b1, w2):
    return pl.pallas_call(
        _tc2_body,
        grid=_grid,
        in_specs=[_pd_spec, _p_spec, _row_spec, _b_spec, _w_spec],
        out_specs=_row_spec,
        out_shape=_rows_out,
    )(pdeg, p1, y1, b1, w2)


def _tc3(pdeg, p2, y2, b2):
    return pl.pallas_call(
        _tc3_body,
        grid=_grid,
        in_specs=[_pd_spec, _p_spec, _row_spec, _b_spec],
        out_specs=_row_spec,
        out_shape=_rows_out,
    )(pdeg, p2, y2, b2)


def kernel(x, edge_index, W1, b1, W2, b2):
    src = edge_index[0].astype(jnp.int32).reshape(_NS, _EC)
    dst = edge_index[1].astype(jnp.int32).reshape(_NS, _EC)
    pad = _ECP - _EC
    srcp = jnp.pad(src, ((0, 0), (0, pad)), constant_values=0)
    dstp = jnp.pad(dst, ((0, 0), (0, pad)), constant_values=_N)
    dstp = dstp.reshape(_NS, _NB, _B)
    xp = jnp.pad(x, ((0, _NP - _N), (0, 0)))

    pdeg = _deg_kernel()(dstp)
    y1 = _tc1(pdeg, xp, W1)
    p1 = _scatter_kernel()(y1.reshape(_NP * 2, _DH), srcp, dstp)
    y2 = _tc2(pdeg, p1, y1, b1.reshape(1, _D), W2)
    p2 = _scatter_kernel()(y2.reshape(_NP * 2, _DH), srcp, dstp)
    out = _tc3(pdeg, p2, y2, b2.reshape(1, _D))
    return out[:_N]


# speedup vs baseline: 1.0037x; 1.0034x over previous
import functools

import jax
import jax.numpy as jnp
from jax import lax
from jax.experimental import pallas as pl
from jax.experimental.pallas import tpu as pltpu
from jax.experimental.pallas import tpu_sc as plsc

_N = 10000
_E = 320000
_D = 128
_DH = _D // 2

_NC = 2
_NS = 16
_EC = _E // _NS
_B = 128
_NB = 160
_ECP = _NB * _B
_NP = 10240
_RPT = _NP // _NS
_DD = 16
_BR = 512


def _mesh():
    return plsc.VectorSubcoreMesh(core_axis_name="c", subcore_axis_name="s")


def _deg_kernel(interpret=False):
    return pl.kernel(
        _deg_body,
        out_type=jax.ShapeDtypeStruct((_NC, _NP, _DD), jnp.float32),
        mesh=_mesh(),
        compiler_params=pltpu.CompilerParams(use_tc_tiling_on_sc=False),
        scratch_types=[
            pltpu.VMEM((_NB, _B), jnp.int32),
            pltpu.VMEM((_B, _DD), jnp.float32),
            pltpu.VMEM((_B, _DD), jnp.float32),
            pltpu.VMEM_SHARED((_NP, _DD), jnp.float32),
        ],
        interpret=interpret,
    )


def _deg_body(dst_hbm, out_hbm, dst_v, ones_v, zero_v, acc):
    c = lax.axis_index("c")
    s = lax.axis_index("s")
    pltpu.sync_copy(dst_hbm.at[s], dst_v)

    ov = jnp.ones((16,), jnp.float32)
    zv = jnp.zeros((16,), jnp.float32)

    def _fill(i, carry):
        ones_v[i, pl.ds(0, _DD)] = ov
        zero_v[i, pl.ds(0, _DD)] = zv
        return carry

    lax.fori_loop(0, _B, _fill, 0)
    rbase = s * _RPT
    for r in range(_RPT // _B):
        pltpu.sync_copy(zero_v, acc.at[pl.ds(rbase + r * _B, _B)])
    plsc.subcore_barrier()

    half = _NB // 2

    def _body(b, carry):
        pltpu.sync_copy(ones_v, acc.at[dst_v.at[c * half + b]], add=True)
        return carry

    lax.fori_loop(0, half, _body, 0)
    plsc.subcore_barrier()
    pltpu.sync_copy(acc.at[pl.ds(rbase, _RPT)], out_hbm.at[c, pl.ds(rbase, _RPT)])


def _scatter_kernel(interpret=False):
    return pl.kernel(
        _scatter_body,
        out_type=jax.ShapeDtypeStruct((_NC, _NP, _DH), jnp.float32),
        mesh=_mesh(),
        compiler_params=pltpu.CompilerParams(use_tc_tiling_on_sc=False),
        scratch_types=[
            pltpu.VMEM((_ECP,), jnp.int32),
            pltpu.VMEM((_NB, _B), jnp.int32),
            pltpu.VMEM((2, 2 * _B, _DH), jnp.float32),
            pltpu.VMEM_SHARED((_NP, _DH), jnp.float32),
            [pltpu.SemaphoreType.DMA] * 2,
            [pltpu.SemaphoreType.DMA] * 2,
        ],
        interpret=interpret,
    )


def _scatter_body(y_hbm, src_hbm, dst_hbm, out_hbm, src_v, dst_v, stage, acc, gsem, ssem):
    c = lax.axis_index("c")
    s = lax.axis_index("s")
    pltpu.sync_copy(src_hbm.at[s], src_v)
    pltpu.sync_copy(dst_hbm.at[s], dst_v)

    def _xform(i, carry):
        v = src_v[pl.ds(i * 16, 16)]
        src_v[pl.ds(i * 16, 16)] = v * 2 + c
        return carry

    lax.fori_loop(0, _ECP // 16, _xform, 0)

    zv = jnp.zeros((16,), jnp.float32)

    def _zrow(i, carry):
        for k in range(2):
            for j in range(_DH // 16):
                stage[k, i, pl.ds(j * 16, 16)] = zv
        return carry

    lax.fori_loop(0, 2 * _B, _zrow, 0)
    rbase = s * _RPT
    for r in range(_RPT // (2 * _B)):
        pltpu.sync_copy(stage.at[0], acc.at[pl.ds(rbase + r * 2 * _B, 2 * _B)])
    rem = _RPT % (2 * _B)
    if rem:
        pltpu.sync_copy(
            stage.at[0, pl.ds(0, rem)], acc.at[pl.ds(rbase + _RPT - rem, rem)]
        )
    plsc.subcore_barrier()

    def _g_start(sb, k):
        bb = sb * 2
        pltpu.make_async_copy(
            y_hbm.at[src_v.at[pl.ds(bb * _B, _B)]],
            stage.at[k, pl.ds(0, _B)], gsem[k],
        ).start()
        pltpu.make_async_copy(
            y_hbm.at[src_v.at[pl.ds((bb + 1) * _B, _B)]],
            stage.at[k, pl.ds(_B, _B)], gsem[k],
        ).start()

    def _g_wait(k):
        pltpu.make_async_copy(
            y_hbm.at[pl.ds(0, 2 * _B)], stage.at[k], gsem[k]
        ).wait()

    def _s_start(sb, k):
        bb = sb * 2
        pltpu.async_copy(
            stage.at[k, pl.ds(0, _B)], acc.at[dst_v.at[bb]], ssem[k],
            add=True)
        pltpu.async_copy(
            stage.at[k, pl.ds(_B, _B)], acc.at[dst_v.at[bb + 1]], ssem[k],
            add=True)

    def _s_wait(k):
        pltpu.make_async_copy(
            y_hbm.at[pl.ds(0, 2 * _B)], stage.at[k], ssem[k]
        ).wait()

    _g_start(0, 0)
    _g_wait(0); _s_start(0, 0); _g_start(1, 1)

    def _body(g, carry):
        sb0 = 2 * g + 1
        _g_wait(1); _s_start(sb0, 1); _s_wait(0); _g_start(sb0 + 1, 0)
        _g_wait(0); _s_start(sb0 + 1, 0); _s_wait(1); _g_start(sb0 + 2, 1)
        return carry

    lax.fori_loop(0, (_NB // 2) // 2 - 1, _body, 0)
    _g_wait(1); _s_start(_NB // 2 - 1, 1); _s_wait(0); _s_wait(1)

    plsc.subcore_barrier()
    pltpu.sync_copy(acc.at[pl.ds(rbase, _RPT)], out_hbm.at[c, pl.ds(rbase, _RPT)])


def _dis_from(pd):
    deg = pd[0, :, 0:1] + pd[1, :, 0:1] + 1.0
    return lax.rsqrt(deg)


def _halves(p):
    return jnp.concatenate((p[0], p[1]), axis=-1)


def _tc1_body(pd_ref, x_ref, w_ref, y_ref):
    dis = _dis_from(pd_ref[...])
    y_ref[...] = jnp.dot(
        x_ref[...], w_ref[...], preferred_element_type=jnp.float32
    ) * dis


def _tc2_body(pd_ref, p_ref, y1_ref, b1_ref, w_ref, y2_ref):
    dis = _dis_from(pd_ref[...])
    pre = (_halves(p_ref[...]) + y1_ref[...]) * dis + b1_ref[...]
    h = jnp.maximum(pre, 0.0)
    y2_ref[...] = jnp.dot(
        h, w_ref[...], preferred_element_type=jnp.float32
    ) * dis


def _tc3_body(pd_ref, p_ref, y2_ref, b2_ref, o_ref):
    dis = _dis_from(pd_ref[...])
    o_ref[...] = (_halves(p_ref[...]) + y2_ref[...]) * dis + b2_ref[...]


_pd_spec = pl.BlockSpec((2, _BR, _DD), lambda i: (0, i, 0))
_row_spec = pl.BlockSpec((_BR, _D), lambda i: (i, 0))
_p_spec = pl.BlockSpec((2, _BR, _DH), lambda i: (0, i, 0))
_w_spec = pl.BlockSpec((_D, _D), lambda i: (0, 0))
_b_spec = pl.BlockSpec((1, _D), lambda i: (0, 0))
_rows_out = jax.ShapeDtypeStruct((_NP, _D), jnp.float32)
_grid = (_NP // _BR,)


def _tc1(pdeg, xp, w1):
    return pl.pallas_call(
        _tc1_body,
        grid=_grid,
        in_specs=[_pd_spec, _row_spec, _w_spec],
        out_specs=_row_spec,
        out_shape=_rows_out,
    )(pdeg, xp, w1)


def _tc2(pdeg, p1, y1, b1, w2):
    return pl.pallas_call(
        _tc2_body,
        grid=_grid,
        in_specs=[_pd_spec, _p_spec, _row_spec, _b_spec, _w_spec],
        out_specs=_row_spec,
        out_shape=_rows_out,
    )(pdeg, p1, y1, b1, w2)


def _tc3(pdeg, p2, y2, b2):
    return pl.pallas_call(
        _tc3_body,
        grid=_grid,
        in_specs=[_pd_spec, _p_spec, _row_spec, _b_spec],
        out_specs=_row_spec,
        out_shape=_rows_out,
    )(pdeg, p2, y2, b2)


def kernel(x, edge_index, W1, b1, W2, b2):
    src = edge_index[0].astype(jnp.int32).reshape(_NS, _EC)
    dst = edge_index[1].astype(jnp.int32).reshape(_NS, _EC)
    pad = _ECP - _EC
    srcp = jnp.pad(src, ((0, 0), (0, pad)), constant_values=0)
    dstp = jnp.pad(dst, ((0, 0), (0, pad)), constant_values=_N)
    dstp = dstp.reshape(_NS, _NB, _B)
    xp = jnp.pad(x, ((0, _NP - _N), (0, 0)))

    pdeg = _deg_kernel()(dstp)
    y1 = _tc1(pdeg, xp, W1)
    p1 = _scatter_kernel()(y1.reshape(_NP * 2, _DH), srcp, dstp)
    y2 = _tc2(pdeg, p1, y1, b1.reshape(1, _D), W2)
    p2 = _scatter_kernel()(y2.reshape(_NP * 2, _DH), srcp, dstp)
    out = _tc3(pdeg, p2, y2, b2.reshape(1, _D))
    return out[:_N]


# speedup vs baseline: 1.0151x; 1.0113x over previous
import functools

import jax
import jax.numpy as jnp
from jax import lax
from jax.experimental import pallas as pl
from jax.experimental.pallas import tpu as pltpu
from jax.experimental.pallas import tpu_sc as plsc

_N = 10000
_E = 320000
_D = 128
_DH = _D // 2

_NC = 2
_NS = 16
_EC = _E // _NS
_B = 128
_NB = 160
_ECP = _NB * _B
_NP = 10240
_RPT = _NP // _NS
_DD = 16
_BR = 512


def _mesh():
    return plsc.VectorSubcoreMesh(core_axis_name="c", subcore_axis_name="s")


def _deg_kernel(interpret=False):
    return pl.kernel(
        _deg_body,
        out_type=jax.ShapeDtypeStruct((_NC, _NP, _DD), jnp.float32),
        mesh=_mesh(),
        compiler_params=pltpu.CompilerParams(use_tc_tiling_on_sc=False),
        scratch_types=[
            pltpu.VMEM((_NB, _B), jnp.int32),
            pltpu.VMEM((_B, _DD), jnp.float32),
            pltpu.VMEM((_B, _DD), jnp.float32),
            pltpu.VMEM_SHARED((_NP, _DD), jnp.float32),
        ],
        interpret=interpret,
    )


def _deg_body(dst_hbm, out_hbm, dst_v, ones_v, zero_v, acc):
    c = lax.axis_index("c")
    s = lax.axis_index("s")
    pltpu.sync_copy(dst_hbm.at[s], dst_v)

    ov = jnp.ones((16,), jnp.float32)
    zv = jnp.zeros((16,), jnp.float32)

    def _fill(i, carry):
        ones_v[i, pl.ds(0, _DD)] = ov
        zero_v[i, pl.ds(0, _DD)] = zv
        return carry

    lax.fori_loop(0, _B, _fill, 0)
    rbase = s * _RPT
    for r in range(_RPT // _B):
        pltpu.sync_copy(zero_v, acc.at[pl.ds(rbase + r * _B, _B)])
    plsc.subcore_barrier()

    half = _NB // 2

    def _body(b, carry):
        pltpu.sync_copy(ones_v, acc.at[dst_v.at[c * half + b]], add=True)
        return carry

    lax.fori_loop(0, half, _body, 0)
    plsc.subcore_barrier()
    pltpu.sync_copy(acc.at[pl.ds(rbase, _RPT)], out_hbm.at[c, pl.ds(rbase, _RPT)])


def _scatter_kernel(interpret=False):
    return pl.kernel(
        _scatter_body,
        out_type=jax.ShapeDtypeStruct((_NC, _NP, _DH), jnp.float32),
        mesh=_mesh(),
        compiler_params=pltpu.CompilerParams(use_tc_tiling_on_sc=False),
        scratch_types=[
            pltpu.VMEM((_ECP,), jnp.int32),
            pltpu.VMEM((_NB, _B), jnp.int32),
            pltpu.VMEM((4, _B, _DH), jnp.float32),
            pltpu.VMEM_SHARED((_NP, _DH), jnp.float32),
            [pltpu.SemaphoreType.DMA] * 4,
            [pltpu.SemaphoreType.DMA] * 4,
        ],
        interpret=interpret,
    )


def _scatter_body(y_hbm, src_hbm, dst_hbm, out_hbm, src_v, dst_v, stage, acc, gsem, ssem):
    c = lax.axis_index("c")
    s = lax.axis_index("s")
    pltpu.sync_copy(src_hbm.at[s], src_v)
    pltpu.sync_copy(dst_hbm.at[s], dst_v)

    def _xform(i, carry):
        v = src_v[pl.ds(i * 16, 16)]
        src_v[pl.ds(i * 16, 16)] = v * 2 + c
        return carry

    lax.fori_loop(0, _ECP // 16, _xform, 0)

    zv = jnp.zeros((16,), jnp.float32)

    def _zrow(i, carry):
        for k in range(4):
            for j in range(_DH // 16):
                stage[k, i, pl.ds(j * 16, 16)] = zv
        return carry

    lax.fori_loop(0, _B, _zrow, 0)
    rbase = s * _RPT
    for r in range(_RPT // _B):
        pltpu.sync_copy(stage.at[0], acc.at[pl.ds(rbase + r * _B, _B)])
    plsc.subcore_barrier()

    def _g_start(bb, k):
        pltpu.make_async_copy(
            y_hbm.at[src_v.at[pl.ds(bb * _B, _B)]], stage.at[k], gsem[k]
        ).start()

    def _g_wait(k):
        pltpu.make_async_copy(
            y_hbm.at[pl.ds(0, _B)], stage.at[k], gsem[k]
        ).wait()

    def _s_start(bb, k):
        pltpu.async_copy(stage.at[k], acc.at[dst_v.at[bb]], ssem[k], add=True)

    def _s_wait(k):
        pltpu.make_async_copy(
            y_hbm.at[pl.ds(0, _B)], stage.at[k], ssem[k]
        ).wait()

    _g_start(0, 0)
    _g_start(1, 1)
    _g_wait(0); _s_start(0, 0); _g_start(2, 2)
    _g_wait(1); _s_start(1, 1); _g_start(3, 3)
    _g_wait(2); _s_start(2, 2); _s_wait(0); _g_start(4, 0)
    _g_wait(3); _s_start(3, 3); _s_wait(1); _g_start(5, 1)

    def _body(g, carry):
        bb = g * 4
        for kk in range(4):
            k2 = (kk + 2) % 4
            _g_wait(kk)
            _s_start(bb + kk, kk)
            _s_wait(k2)
            _g_start(bb + kk + 2, k2)
        return carry

    lax.fori_loop(1, _NB // 4 - 1, _body, 0)
    t = _NB - 4
    _g_wait(0); _s_start(t + 0, 0); _s_wait(2); _g_start(t + 2, 2)
    _g_wait(1); _s_start(t + 1, 1); _s_wait(3); _g_start(t + 3, 3)
    _g_wait(2); _s_start(t + 2, 2); _s_wait(0)
    _g_wait(3); _s_start(t + 3, 3); _s_wait(1)
    _s_wait(2)
    _s_wait(3)

    plsc.subcore_barrier()
    pltpu.sync_copy(acc.at[pl.ds(rbase, _RPT)], out_hbm.at[c, pl.ds(rbase, _RPT)])


def _dis_from(pd):
    deg = pd[0, :, 0:1] + pd[1, :, 0:1] + 1.0
    return lax.rsqrt(deg)


def _halves(p):
    return jnp.concatenate((p[0], p[1]), axis=-1)


def _tc1_body(pd_ref, x_ref, w_ref, y_ref):
    dis = _dis_from(pd_ref[...])
    y_ref[...] = jnp.dot(
        x_ref[...], w_ref[...], preferred_element_type=jnp.float32
    ) * dis


def _tc2_body(pd_ref, p_ref, y1_ref, b1_ref, w_ref, y2_ref):
    dis = _dis_from(pd_ref[...])
    pre = (_halves(p_ref[...]) + y1_ref[...]) * dis + b1_ref[...]
    h = jnp.maximum(pre, 0.0)
    y2_ref[...] = jnp.dot(
        h, w_ref[...], preferred_element_type=jnp.float32
    ) * dis


def _tc3_body(pd_ref, p_ref, y2_ref, b2_ref, o_ref):
    dis = _dis_from(pd_ref[...])
    o_ref[...] = (_halves(p_ref[...]) + y2_ref[...]) * dis + b2_ref[...]


_pd_spec = pl.BlockSpec((2, _BR, _DD), lambda i: (0, i, 0))
_row_spec = pl.BlockSpec((_BR, _D), lambda i: (i, 0))
_p_spec = pl.BlockSpec((2, _BR, _DH), lambda i: (0, i, 0))
_w_spec = pl.BlockSpec((_D, _D), lambda i: (0, 0))
_b_spec = pl.BlockSpec((1, _D), lambda i: (0, 0))
_rows_out = jax.ShapeDtypeStruct((_NP, _D), jnp.float32)
_grid = (_NP // _BR,)


def _tc1(pdeg, xp, w1):
    return pl.pallas_call(
        _tc1_body,
        grid=_grid,
        in_specs=[_pd_spec, _row_spec, _w_spec],
        out_specs=_row_spec,
        out_shape=_rows_out,
    )(pdeg, xp, w1)


def _tc2(pdeg, p1, y1, b1, w2):
    return pl.pallas_call(
        _tc2_body,
        grid=_grid,
        in_specs=[_pd_spec, _p_spec, _row_spec, _b_spec, _w_spec],
        out_specs=_row_spec,
        out_shape=_rows_out,
    )(pdeg, p1, y1, b1, w2)


def _tc3(pdeg, p2, y2, b2):
    return pl.pallas_call(
        _tc3_body,
        grid=_grid,
        in_specs=[_pd_spec, _p_spec, _row_spec, _b_spec],
        out_specs=_row_spec,
        out_shape=_rows_out,
    )(pdeg, p2, y2, b2)


def kernel(x, edge_index, W1, b1, W2, b2):
    src = edge_index[0].astype(jnp.int32).reshape(_NS, _EC)
    dst = edge_index[1].astype(jnp.int32).reshape(_NS, _EC)
    pad = _ECP - _EC
    srcp = jnp.pad(src, ((0, 0), (0, pad)), constant_values=0)
    dstp = jnp.pad(dst, ((0, 0), (0, pad)), constant_values=_N)
    dstp = dstp.reshape(_NS, _NB, _B)
    xp = jnp.pad(x, ((0, _NP - _N), (0, 0)))

    pdeg = _deg_kernel()(dstp)
    y1 = _tc1(pdeg, xp, W1)
    p1 = _scatter_kernel()(y1.reshape(_NP * 2, _DH), srcp, dstp)
    y2 = _tc2(pdeg, p1, y1, b1.reshape(1, _D), W2)
    p2 = _scatter_kernel()(y2.reshape(_NP * 2, _DH), srcp, dstp)
    out = _tc3(pdeg, p2, y2, b2.reshape(1, _D))
    return out[:_N]


# speedup vs baseline: 1.0153x; 1.0002x over previous
import jax
import jax.numpy as jnp
from jax import lax
from jax.experimental import pallas as pl
from jax.experimental.pallas import tpu as pltpu
from jax.experimental.pallas import tpu_sc as plsc

_N = 10000
_E = 320000
_D = 128
_DH = _D // 2

_NC = 2
_NS = 16
_EC = _E // _NS
_B = 128
_NB = 160
_ECP = _NB * _B
_NP = 10240
_RPT = _NP // _NS
_DD = 16
_BR = 512


def _mesh():
    return plsc.VectorSubcoreMesh(core_axis_name="c", subcore_axis_name="s")


def _deg_kernel():
    return pl.kernel(
        _deg_body,
        out_type=jax.ShapeDtypeStruct((_NC, _NP, _DD), jnp.float32),
        mesh=_mesh(),
        compiler_params=pltpu.CompilerParams(use_tc_tiling_on_sc=False),
        scratch_types=[
            pltpu.VMEM((_NB, _B), jnp.int32),
            pltpu.VMEM((_B, _DD), jnp.float32),
            pltpu.VMEM((_B, _DD), jnp.float32),
            pltpu.VMEM_SHARED((_NP, _DD), jnp.float32),
        ],
    )


def _deg_body(dst_hbm, out_hbm, dst_v, ones_v, zero_v, acc):
    c = lax.axis_index("c")
    s = lax.axis_index("s")
    pltpu.sync_copy(dst_hbm.at[s], dst_v)

    ov = jnp.ones((16,), jnp.float32)
    zv = jnp.zeros((16,), jnp.float32)

    def _fill(i, carry):
        ones_v[i, pl.ds(0, _DD)] = ov
        zero_v[i, pl.ds(0, _DD)] = zv
        return carry

    lax.fori_loop(0, _B, _fill, 0)
    rbase = s * _RPT
    for r in range(_RPT // _B):
        pltpu.sync_copy(zero_v, acc.at[pl.ds(rbase + r * _B, _B)])
    plsc.subcore_barrier()

    half = _NB // 2

    def _body(b, carry):
        pltpu.sync_copy(ones_v, acc.at[dst_v.at[c * half + b]], add=True)
        return carry

    lax.fori_loop(0, half, _body, 0)
    plsc.subcore_barrier()
    pltpu.sync_copy(acc.at[pl.ds(rbase, _RPT)], out_hbm.at[c, pl.ds(rbase, _RPT)])


def _scatter_kernel():
    return pl.kernel(
        _scatter_body,
        out_type=jax.ShapeDtypeStruct((_NC, _NP, _DH), jnp.float32),
        mesh=_mesh(),
        compiler_params=pltpu.CompilerParams(use_tc_tiling_on_sc=False),
        scratch_types=[
            pltpu.VMEM((_ECP,), jnp.int32),
            pltpu.VMEM((_NB, _B), jnp.int32),
            pltpu.VMEM((4, _B, _DH), jnp.float32),
            pltpu.VMEM_SHARED((_NP, _DH), jnp.float32),
            [pltpu.SemaphoreType.DMA] * 4,
            [pltpu.SemaphoreType.DMA] * 4,
        ],
    )


def _scatter_body(y_hbm, src_hbm, dst_hbm, out_hbm, src_v, dst_v, stage, acc, gsem, ssem):
    c = lax.axis_index("c")
    s = lax.axis_index("s")
    pltpu.sync_copy(src_hbm.at[s], src_v)
    pltpu.sync_copy(dst_hbm.at[s], dst_v)

    def _xform(i, carry):
        v = src_v[pl.ds(i * 16, 16)]
        src_v[pl.ds(i * 16, 16)] = v * 2 + c
        return carry

    lax.fori_loop(0, _ECP // 16, _xform, 0)

    zv = jnp.zeros((16,), jnp.float32)

    def _zrow(i, carry):
        for k in range(4):
            for j in range(_DH // 16):
                stage[k, i, pl.ds(j * 16, 16)] = zv
        return carry

    lax.fori_loop(0, _B, _zrow, 0)
    rbase = s * _RPT
    for r in range(_RPT // _B):
        pltpu.sync_copy(stage.at[0], acc.at[pl.ds(rbase + r * _B, _B)])
    plsc.subcore_barrier()

    def _g_start(bb, k):
        pltpu.make_async_copy(
            y_hbm.at[src_v.at[pl.ds(bb * _B, _B)]], stage.at[k], gsem[k]
        ).start()

    def _g_wait(k):
        pltpu.make_async_copy(
            y_hbm.at[pl.ds(0, _B)], stage.at[k], gsem[k]
        ).wait()

    def _s_start(bb, k):
        pltpu.async_copy(stage.at[k], acc.at[dst_v.at[bb]], ssem[k], add=True)

    def _s_wait(k):
        pltpu.make_async_copy(
            y_hbm.at[pl.ds(0, _B)], stage.at[k], ssem[k]
        ).wait()

    _g_start(0, 0)
    _g_start(1, 1)
    _g_wait(0); _s_start(0, 0); _g_start(2, 2)
    _g_wait(1); _s_start(1, 1); _g_start(3, 3)
    _g_wait(2); _s_start(2, 2); _s_wait(0); _g_start(4, 0)
    _g_wait(3); _s_start(3, 3); _s_wait(1); _g_start(5, 1)

    def _body(g, carry):
        bb = g * 4
        for kk in range(4):
            k2 = (kk + 2) % 4
            _g_wait(kk)
            _s_start(bb + kk, kk)
            _s_wait(k2)
            _g_start(bb + kk + 2, k2)
        return carry

    lax.fori_loop(1, _NB // 4 - 1, _body, 0)
    t = _NB - 4
    _g_wait(0); _s_start(t + 0, 0); _s_wait(2); _g_start(t + 2, 2)
    _g_wait(1); _s_start(t + 1, 1); _s_wait(3); _g_start(t + 3, 3)
    _g_wait(2); _s_start(t + 2, 2); _s_wait(0)
    _g_wait(3); _s_start(t + 3, 3); _s_wait(1)
    _s_wait(2)
    _s_wait(3)

    plsc.subcore_barrier()
    pltpu.sync_copy(acc.at[pl.ds(rbase, _RPT)], out_hbm.at[c, pl.ds(rbase, _RPT)])


def _dis_from(pd):
    deg = pd[0, :, 0:1] + pd[1, :, 0:1] + 1.0
    return lax.rsqrt(deg)


def _halves(p):
    return jnp.concatenate((p[0], p[1]), axis=-1)


def _tc1_body(pd_ref, x_ref, w_ref, y_ref):
    dis = _dis_from(pd_ref[...])
    y_ref[...] = jnp.dot(
        x_ref[...], w_ref[...], preferred_element_type=jnp.float32
    ) * dis


def _tc2_body(pd_ref, p_ref, y1_ref, b1_ref, w_ref, y2_ref):
    dis = _dis_from(pd_ref[...])
    pre = (_halves(p_ref[...]) + y1_ref[...]) * dis + b1_ref[...]
    h = jnp.maximum(pre, 0.0)
    y2_ref[...] = jnp.dot(
        h, w_ref[...], preferred_element_type=jnp.float32
    ) * dis


def _tc3_body(pd_ref, p_ref, y2_ref, b2_ref, o_ref):
    dis = _dis_from(pd_ref[...])
    o_ref[...] = (_halves(p_ref[...]) + y2_ref[...]) * dis + b2_ref[...]


_pd_spec = pl.BlockSpec((2, _BR, _DD), lambda i: (0, i, 0))
_row_spec = pl.BlockSpec((_BR, _D), lambda i: (i, 0))
_p_spec = pl.BlockSpec((2, _BR, _DH), lambda i: (0, i, 0))
_w_spec = pl.BlockSpec((_D, _D), lambda i: (0, 0))
_b_spec = pl.BlockSpec((1, _D), lambda i: (0, 0))
_rows_out = jax.ShapeDtypeStruct((_NP, _D), jnp.float32)
_grid = (_NP // _BR,)


def _tc1(pdeg, xp, w1):
    return pl.pallas_call(
        _tc1_body,
        grid=_grid,
        in_specs=[_pd_spec, _row_spec, _w_spec],
        out_specs=_row_spec,
        out_shape=_rows_out,
    )(pdeg, xp, w1)


def _tc2(pdeg, p1, y1, b1, w2):
    return pl.pallas_call(
        _tc2_body,
        grid=_grid,
        in_specs=[_pd_spec, _p_spec, _row_spec, _b_spec, _w_spec],
        out_specs=_row_spec,
        out_shape=_rows_out,
    )(pdeg, p1, y1, b1, w2)


def _tc3(pdeg, p2, y2, b2):
    return pl.pallas_call(
        _tc3_body,
        grid=_grid,
        in_specs=[_pd_spec, _p_spec, _row_spec, _b_spec],
        out_specs=_row_spec,
        out_shape=_rows_out,
    )(pdeg, p2, y2, b2)


def kernel(x, edge_index, W1, b1, W2, b2):
    src = edge_index[0].astype(jnp.int32).reshape(_NS, _EC)
    dst = edge_index[1].astype(jnp.int32).reshape(_NS, _EC)
    pad = _ECP - _EC
    srcp = jnp.pad(src, ((0, 0), (0, pad)), constant_values=0)
    dstp = jnp.pad(dst, ((0, 0), (0, pad)), constant_values=_N)
    dstp = dstp.reshape(_NS, _NB, _B)
    xp = jnp.pad(x, ((0, _NP - _N), (0, 0)))

    pdeg = _deg_kernel()(dstp)
    y1 = _tc1(pdeg, xp, W1)
    p1 = _scatter_kernel()(y1.reshape(_NP * 2, _DH), srcp, dstp)
    y2 = _tc2(pdeg, p1, y1, b1.reshape(1, _D), W2)
    p2 = _scatter_kernel()(y2.reshape(_NP * 2, _DH), srcp, dstp)
    out = _tc3(pdeg, p2, y2, b2.reshape(1, _D))
    return out[:_N]
